# baseline (speedup 1.0000x reference)
import functools

import jax
import jax.numpy as jnp
from jax import lax
from jax.experimental import pallas as pl
from jax.experimental.pallas import tpu as pltpu
from jax.experimental.pallas import tpu_sc as plsc

B = 16384
RF2 = 121
CD = 64
FD = RF2 * 16
AD = 2
OUT = CD + FD + AD
OUTP = 2048
NC, NS = 2, 16
NW = NC * NS
R = B // NW
C = 8
NCHUNK = R // C

_mesh = plsc.VectorSubcoreMesh(core_axis_name="c", subcore_axis_name="s")


@functools.partial(
    pl.kernel,
    out_type=jax.ShapeDtypeStruct((B, OUTP), jnp.float32),
    mesh=_mesh,
    compiler_params=pltpu.CompilerParams(use_tc_tiling_on_sc=False,
                                         needs_layout_passes=False),
    scratch_types=[
        pltpu.VMEM((2, C * RF2), jnp.int32),
        pltpu.VMEM((R * 4,), jnp.int32),
        pltpu.VMEM((R,), jnp.int32),
        pltpu.VMEM((2, C * RF2, 16), jnp.float32),
        pltpu.VMEM((2, C * 4, 16), jnp.float32),
        pltpu.VMEM((2, C, OUTP), jnp.float32),
        pltpu.VMEM((8,), jnp.float32),
        pltpu.SemaphoreType.DMA,
        pltpu.SemaphoreType.DMA,
        pltpu.SemaphoreType.DMA,
        pltpu.SemaphoreType.DMA,
        pltpu.SemaphoreType.DMA,
        pltpu.SemaphoreType.DMA,
        pltpu.SemaphoreType.DMA,
    ],
)
def _hint_kernel(w16, wf, wa, cidx_hbm, fidx_hbm, act_hbm, out,
                 fidx_v, cidx_v, act_v, fbuf, cbuf, obuf, wa_v,
                 semf0, semf1, semc0, semc1, semw0, semw1, sems):
    wid = lax.axis_index("s") * NC + lax.axis_index("c")
    rbase = wid * R
    pltpu.sync_copy(wa, wa_v)
    pltpu.sync_copy(cidx_hbm.at[pl.ds(rbase * 4, R * 4)], cidx_v)
    pltpu.sync_copy(act_hbm.at[pl.ds(rbase, R)], act_v)

    semf = (semf0, semf1)
    semc = (semc0, semc1)
    semw = (semw0, semw1)

    def fire(g, s, guard=False):
        def _go():
            base = rbase + g * C
            pltpu.async_copy(fidx_hbm.at[pl.ds(base * RF2, C * RF2)],
                             fidx_v.at[s], sems).wait()
            pltpu.make_async_copy(wf.at[fidx_v.at[s]], fbuf.at[s],
                                  semf[s]).start()
            pltpu.make_async_copy(w16.at[cidx_v.at[pl.ds(g * C * 4, C * 4)]],
                                  cbuf.at[s], semc[s]).start()
        if guard:
            pl.when(g < NCHUNK)(_go)
        else:
            _go()

    def process(g, s, first):
        base = rbase + g * C
        pltpu.make_async_copy(wf.at[fidx_v.at[s]], fbuf.at[s], semf[s]).wait()
        pltpu.make_async_copy(w16.at[cidx_v.at[pl.ds(g * C * 4, C * 4)]],
                              cbuf.at[s], semc[s]).wait()
        def _drain():
            pltpu.make_async_copy(obuf.at[s], out.at[pl.ds(base, C), :],
                                  semw[s]).wait()
        if first:
            pl.when(g >= 2)(_drain)
        else:
            _drain()

        for r in range(C):
            for j in range(4):
                obuf[s, r, pl.ds(16 * j, 16)] = cbuf[s, r * 4 + j, :]
            for j in range(RF2):
                obuf[s, r, pl.ds(CD + 16 * j, 16)] = fbuf[s, r * RF2 + j, :]

        lanes = lax.iota(jnp.int32, 16)
        rows = lanes // 2
        cols = lanes % 2
        a = plsc.load_gather(act_v, [g * C + rows])
        w = plsc.load_gather(wa_v, [a * 2 + cols])
        plsc.store_scatter(obuf.at[s], [rows, cols + (CD + FD)], w)

        pltpu.make_async_copy(obuf.at[s], out.at[pl.ds(base, C), :],
                              semw[s]).start()

    fire(0, 0)

    @pl.loop(0, NCHUNK // 2)
    def _pair(t):
        g0 = 2 * t
        fire(g0 + 1, 1)
        process(g0, 0, first=True)
        fire(g0 + 2, 0, guard=True)
        process(g0 + 1, 1, first=True)

    pltpu.make_async_copy(obuf.at[0], out.at[pl.ds(rbase, C), :], semw0).wait()
    pltpu.make_async_copy(obuf.at[1], out.at[pl.ds(rbase, C), :], semw1).wait()


CW = B // NW
HW = CW // 4


@functools.partial(
    pl.kernel,
    out_type=jax.ShapeDtypeStruct((B * RF2,), jnp.int32),
    mesh=_mesh,
    compiler_params=pltpu.CompilerParams(use_tc_tiling_on_sc=False,
                                         needs_layout_passes=False),
    scratch_types=[
        pltpu.VMEM((RF2, HW), jnp.int32),
        pltpu.VMEM((HW * RF2,), jnp.int32),
        pltpu.SemaphoreType.DMA,
    ],
)
def _prep_idx(fidx_t, fidx_rm, iv, ibuf, semi):
    wid = lax.axis_index("s") * NC + lax.axis_index("c")
    iota = lax.iota(jnp.int32, 16)
    iota121 = iota * RF2
    for h in range(4):
        cb0 = wid * CW + h * HW
        pltpu.async_copy(fidx_t.at[:, pl.ds(cb0, HW)], iv, semi).wait()

        @pl.loop(0, HW // 16)
        def _cb(cb):
            rows16 = iota121 + cb * (16 * RF2)
            for p in range(RF2):
                v = iv[p, pl.ds(16 * cb, 16)]
                plsc.store_scatter(ibuf, [rows16 + p], v)

        pltpu.async_copy(ibuf, fidx_rm.at[pl.ds(cb0 * RF2, HW * RF2)],
                         semi).wait()


def kernel(coords, obses, actions, W_coord, W_field, W_action):
    c2 = coords.astype(jnp.int32) * 2
    cidx = jnp.stack([c2[:, 0], c2[:, 0] + 1, c2[:, 1], c2[:, 1] + 1],
                     axis=1).reshape(-1)
    fidx_t = obses.astype(jnp.int32).transpose(1, 2, 0).reshape(RF2, B)
    fidx = _prep_idx(fidx_t)
    act = actions.astype(jnp.int32).reshape(-1)
    w16 = W_coord.reshape(2000, 16)
    wa = W_action.reshape(-1)
    return _hint_kernel(w16, W_field, wa, cidx, fidx, act)[:, :OUT]


# speedup vs baseline: 1.0275x; 1.0275x over previous
import functools

import jax
import jax.numpy as jnp
from jax import lax
from jax.experimental import pallas as pl
from jax.experimental.pallas import tpu as pltpu
from jax.experimental.pallas import tpu_sc as plsc

B = 16384
RF2 = 121
CD = 64
FD = RF2 * 16
AD = 2
OUT = CD + FD + AD
NC, NS = 2, 16
NW = NC * NS
R = B // NW
C = 8
NCHUNK = R // C

_mesh = plsc.VectorSubcoreMesh(core_axis_name="c", subcore_axis_name="s")


@functools.partial(
    pl.kernel,
    out_type=jax.ShapeDtypeStruct((B, OUT), jnp.float32),
    mesh=_mesh,
    compiler_params=pltpu.CompilerParams(use_tc_tiling_on_sc=False,
                                         needs_layout_passes=False),
    scratch_types=[
        pltpu.VMEM((2, C * RF2), jnp.int32),
        pltpu.VMEM((R * 4,), jnp.int32),
        pltpu.VMEM((R,), jnp.int32),
        pltpu.VMEM((2, C * RF2, 16), jnp.float32),
        pltpu.VMEM((2, C * 4, 16), jnp.float32),
        pltpu.VMEM((2, C, OUT), jnp.float32),
        pltpu.VMEM((8,), jnp.float32),
        pltpu.SemaphoreType.DMA,
        pltpu.SemaphoreType.DMA,
        pltpu.SemaphoreType.DMA,
        pltpu.SemaphoreType.DMA,
        pltpu.SemaphoreType.DMA,
        pltpu.SemaphoreType.DMA,
        pltpu.SemaphoreType.DMA,
        pltpu.SemaphoreType.DMA,
    ],
)
def _hint_kernel(w16, wf, wa, cidx_hbm, fidx_hbm, act_hbm, out,
                 fidx_v, cidx_v, act_v, fbuf, cbuf, obuf, wa_v,
                 semf0, semf1, semc0, semc1, semw0, semw1, semi0, semi1):
    wid = lax.axis_index("s") * NC + lax.axis_index("c")
    rbase = wid * R
    pltpu.sync_copy(wa, wa_v)
    pltpu.sync_copy(cidx_hbm.at[pl.ds(rbase * 4, R * 4)], cidx_v)
    pltpu.sync_copy(act_hbm.at[pl.ds(rbase, R)], act_v)

    semf = (semf0, semf1)
    semc = (semc0, semc1)
    semw = (semw0, semw1)
    semi = (semi0, semi1)

    def load_idx(g, s, guard=False):
        def _go():
            base = rbase + g * C
            pltpu.make_async_copy(fidx_hbm.at[pl.ds(base * RF2, C * RF2)],
                                  fidx_v.at[s], semi[s]).start()
        if guard:
            pl.when(g < NCHUNK)(_go)
        else:
            _go()

    def fire(g, s, guard=False):
        def _go():
            base = rbase + g * C
            pltpu.make_async_copy(fidx_hbm.at[pl.ds(base * RF2, C * RF2)],
                                  fidx_v.at[s], semi[s]).wait()
            pltpu.make_async_copy(wf.at[fidx_v.at[s]], fbuf.at[s],
                                  semf[s]).start()
            pltpu.make_async_copy(w16.at[cidx_v.at[pl.ds(g * C * 4, C * 4)]],
                                  cbuf.at[s], semc[s]).start()
        if guard:
            pl.when(g < NCHUNK)(_go)
        else:
            _go()

    def process(g, s, first):
        base = rbase + g * C
        pltpu.make_async_copy(wf.at[fidx_v.at[s]], fbuf.at[s], semf[s]).wait()
        pltpu.make_async_copy(w16.at[cidx_v.at[pl.ds(g * C * 4, C * 4)]],
                              cbuf.at[s], semc[s]).wait()
        load_idx(g + 2, s, guard=True)
        def _drain():
            pltpu.make_async_copy(obuf.at[s], out.at[pl.ds(base, C), :],
                                  semw[s]).wait()
        if first:
            pl.when(g >= 2)(_drain)
        else:
            _drain()

        for r in range(C):
            for j in range(4):
                obuf[s, r, pl.ds(16 * j, 16)] = cbuf[s, r * 4 + j, :]
            for j in range(RF2):
                obuf[s, r, pl.ds(CD + 16 * j, 16)] = fbuf[s, r * RF2 + j, :]

        lanes = lax.iota(jnp.int32, 16)
        rows = lanes // 2
        cols = lanes % 2
        a = plsc.load_gather(act_v, [g * C + rows])
        w = plsc.load_gather(wa_v, [a * 2 + cols])
        plsc.store_scatter(obuf.at[s], [rows, cols + (CD + FD)], w)

        pltpu.make_async_copy(obuf.at[s], out.at[pl.ds(base, C), :],
                              semw[s]).start()

    load_idx(0, 0)
    load_idx(1, 1)
    fire(0, 0)

    @pl.loop(0, NCHUNK // 2)
    def _pair(t):
        g0 = 2 * t
        fire(g0 + 1, 1)
        process(g0, 0, first=True)
        fire(g0 + 2, 0, guard=True)
        process(g0 + 1, 1, first=True)

    pltpu.make_async_copy(obuf.at[0], out.at[pl.ds(rbase, C), :], semw0).wait()
    pltpu.make_async_copy(obuf.at[1], out.at[pl.ds(rbase, C), :], semw1).wait()


CW = B // NW
HW = CW // 4


@functools.partial(
    pl.kernel,
    out_type=jax.ShapeDtypeStruct((B * RF2,), jnp.int32),
    mesh=_mesh,
    compiler_params=pltpu.CompilerParams(use_tc_tiling_on_sc=False,
                                         needs_layout_passes=False),
    scratch_types=[
        pltpu.VMEM((RF2, HW), jnp.int32),
        pltpu.VMEM((HW * RF2,), jnp.int32),
        pltpu.SemaphoreType.DMA,
    ],
)
def _prep_idx(fidx_t, fidx_rm, iv, ibuf, semi):
    wid = lax.axis_index("s") * NC + lax.axis_index("c")
    iota = lax.iota(jnp.int32, 16)
    iota121 = iota * RF2
    for h in range(4):
        cb0 = wid * CW + h * HW
        pltpu.async_copy(fidx_t.at[:, pl.ds(cb0, HW)], iv, semi).wait()

        @pl.loop(0, HW // 16)
        def _cb(cb):
            rows16 = iota121 + cb * (16 * RF2)
            for p in range(RF2):
                v = iv[p, pl.ds(16 * cb, 16)]
                plsc.store_scatter(ibuf, [rows16 + p], v)

        pltpu.async_copy(ibuf, fidx_rm.at[pl.ds(cb0 * RF2, HW * RF2)],
                         semi).wait()


def kernel(coords, obses, actions, W_coord, W_field, W_action):
    c2 = coords.astype(jnp.int32) * 2
    cidx = jnp.stack([c2[:, 0], c2[:, 0] + 1, c2[:, 1], c2[:, 1] + 1],
                     axis=1).reshape(-1)
    fidx_t = obses.astype(jnp.int32).transpose(1, 2, 0).reshape(RF2, B)
    fidx = _prep_idx(fidx_t)
    act = actions.astype(jnp.int32).reshape(-1)
    w16 = W_coord.reshape(2000, 16)
    wa = W_action.reshape(-1)
    return _hint_kernel(w16, W_field, wa, cidx, fidx, act)
